# baseline (speedup 1.0000x reference)
import jax
import jax.numpy as jnp
from jax.experimental import pallas as pl

NUM_HEADS = 8
OUT_FEATS = 64
CHUNK = 2000


def _proj_kernel(x_ref, w_ref, o_ref):
    w = w_ref[:]
    for c in range(x_ref.shape[0] // CHUNK):
        acc = jnp.dot(x_ref[pl.ds(c * CHUNK, CHUNK), :], w,
                      preferred_element_type=jnp.float32)
        o_ref[pl.ds(c * CHUNK, CHUNK), :, :] = acc.reshape(CHUNK, NUM_HEADS, OUT_FEATS)


def kernel(feat, edge_index, W_fc_self):
    del edge_index
    n, in_feats = feat.shape
    m = W_fc_self.shape[0]
    wt = W_fc_self.T
    out = pl.pallas_call(
        _proj_kernel,
        in_specs=[
            pl.BlockSpec((n, in_feats), lambda: (0, 0)),
            pl.BlockSpec((in_feats, m), lambda: (0, 0)),
        ],
        out_specs=pl.BlockSpec((n, NUM_HEADS, OUT_FEATS), lambda: (0, 0, 0)),
        out_shape=jax.ShapeDtypeStruct((n, NUM_HEADS, OUT_FEATS), jnp.float32),
    )(feat, wt)
    return out


# speedup vs baseline: 1.8380x; 1.8380x over previous
import jax
import jax.numpy as jnp
from jax.experimental import pallas as pl

NUM_HEADS = 8
OUT_FEATS = 64
ROW_BLOCK = 1000


def _proj_kernel(x_ref, w_ref, o_ref):
    acc = jnp.dot(x_ref[:], w_ref[:], preferred_element_type=jnp.float32)
    o_ref[:] = acc.astype(jnp.bfloat16)


def kernel(feat, edge_index, W_fc_self):
    del edge_index
    n, in_feats = feat.shape
    m = W_fc_self.shape[0]
    wt = W_fc_self.T
    out = pl.pallas_call(
        _proj_kernel,
        grid=(n // ROW_BLOCK,),
        in_specs=[
            pl.BlockSpec((ROW_BLOCK, in_feats), lambda i: (i, 0)),
            pl.BlockSpec((in_feats, m), lambda i: (0, 0)),
        ],
        out_specs=pl.BlockSpec((ROW_BLOCK, m), lambda i: (i, 0)),
        out_shape=jax.ShapeDtypeStruct((n, m), jnp.bfloat16),
    )(feat, wt)
    return out.astype(jnp.float32).reshape(n, NUM_HEADS, OUT_FEATS)
